# baseline (speedup 1.0000x reference)
import jax
import jax.numpy as jnp
from jax import lax
from jax.experimental import pallas as pl
from jax.experimental.pallas import tpu as pltpu
from jax.experimental.pallas import tpu_sc as plsc

N = 10000
D = 128
E = 320000
NC = 2
NS = 16
NW = NC * NS
CH = 112
NCHUNK = 92
EPW = CH * NCHUNK
EPAD = NW * EPW
NBUF = 2
PACK = 16384
NZB = N // CH
NREM = N - NZB * CH


def _spmm_body(emb_hbm, pack_hbm, vals_hbm, out_hbm,
               pack_v, vals_v, rows_v, srci_v, dsti_v, acc_sh,
               g0, g1, s0, s1):
    gsem = (g0, g1)
    ssem = (s0, s1)
    cid = lax.axis_index("c")
    sid = lax.axis_index("s")
    wid = cid * NS + sid
    ebase = wid * EPW

    zeros16 = jnp.zeros((16,), jnp.float32)
    zbuf = rows_v.at[0]

    @pl.loop(0, CH)
    def _zero_rows(i):
        for j in range(D // 16):
            zbuf[i, pl.ds(j * 16, 16)] = zeros16

    @pl.loop(sid, NZB, step=NS)
    def _zero_acc(cz):
        pltpu.sync_copy(zbuf, acc_sh.at[pl.ds(cz * CH, CH)])

    @pl.when(sid == NS - 1)
    def _zero_rem():
        pltpu.sync_copy(rows_v.at[0, pl.ds(0, NREM)],
                        acc_sh.at[pl.ds(NZB * CH, NREM)])

    plsc.subcore_barrier()

    pltpu.sync_copy(pack_hbm.at[pl.ds(ebase, EPW)], pack_v)
    pltpu.sync_copy(vals_hbm.at[pl.ds(ebase, EPW)], vals_v)

    def unpack_idx(ch, b):
        for m in range(CH // 16):
            v = pack_v[pl.ds(ch * CH + m * 16, 16)]
            srci_v[b, pl.ds(m * 16, 16)] = v & (PACK - 1)
            dsti_v[b, pl.ds(m * 16, 16)] = jnp.right_shift(v, 14)

    def issue_gather(ch, b):
        pltpu.async_copy(emb_hbm.at[srci_v.at[b]], rows_v.at[b], gsem[b])

    def wait_gather(b):
        pltpu.make_async_copy(emb_hbm.at[pl.ds(0, CH)], rows_v.at[b],
                              gsem[b]).wait()

    def wait_scatter(b):
        pltpu.make_async_copy(rows_v.at[b], acc_sh.at[pl.ds(0, CH)],
                              ssem[b]).wait()

    unpack_idx(0, 0)
    issue_gather(0, 0)

    @pl.loop(0, NCHUNK, step=NBUF)
    def _chunks(g):
        for b in range(NBUF):
            ch = g + b
            b1 = (b + 1) % NBUF

            @pl.when(ch + 1 < NCHUNK)
            def _prefetch():
                @pl.when(ch >= 1)
                def _drain():
                    wait_scatter(b1)

                unpack_idx(ch + 1, b1)
                issue_gather(ch + 1, b1)

            wait_gather(b)
            rowsb = rows_v.at[b]

            @pl.loop(0, CH // 16)
            def _scale(grp):
                v16 = vals_v[pl.ds(ch * CH + grp * 16, 16)]
                for r in range(16):
                    v = v16[r]
                    row = grp * 16 + r
                    for j in range(D // 16):
                        rowsb[row, pl.ds(j * 16, 16)] = (
                            rowsb[row, pl.ds(j * 16, 16)] * v)

            pltpu.async_copy(rows_v.at[b], acc_sh.at[dsti_v.at[b]],
                             ssem[b], add=True)

    for k in range(NBUF):
        wait_scatter((NCHUNK - NBUF + k) % NBUF)

    plsc.subcore_barrier()

    @pl.loop(sid, NZB, step=NS)
    def _writeout(co):
        r0 = co * CH
        pltpu.sync_copy(acc_sh.at[pl.ds(r0, CH)], zbuf)
        pltpu.sync_copy(zbuf, out_hbm.at[pl.ds(cid * N + r0, CH)])

    @pl.when(sid == NS - 1)
    def _writeout_rem():
        pltpu.sync_copy(acc_sh.at[pl.ds(NZB * CH, NREM)],
                        rows_v.at[0, pl.ds(0, NREM)])
        pltpu.sync_copy(rows_v.at[0, pl.ds(0, NREM)],
                        out_hbm.at[pl.ds(cid * N + NZB * CH, NREM)])


_spmm_sc = pl.kernel(
    _spmm_body,
    out_type=jax.ShapeDtypeStruct((NC * N, D), jnp.float32),
    mesh=plsc.VectorSubcoreMesh(core_axis_name="c", subcore_axis_name="s"),
    scratch_types=[
        pltpu.VMEM((EPW,), jnp.int32),
        pltpu.VMEM((EPW,), jnp.float32),
        pltpu.VMEM((NBUF, CH, D), jnp.float32),
        pltpu.VMEM((NBUF, CH), jnp.int32),
        pltpu.VMEM((NBUF, CH), jnp.int32),
        pltpu.VMEM_SHARED((N, D), jnp.float32),
        pltpu.SemaphoreType.DMA, pltpu.SemaphoreType.DMA,
        pltpu.SemaphoreType.DMA, pltpu.SemaphoreType.DMA,
    ],
)


def _dense_body(p_ref, wg_ref, bg_ref, wb_ref, bb_ref, emb_ref, mlp_ref):
    s = p_ref[0] + p_ref[1]
    h = jnp.dot(s, wg_ref[...], preferred_element_type=jnp.float32)
    h = h + bg_ref[...]
    h = jnp.where(h >= 0, h, 0.01 * h)
    emb_ref[...] = h
    mlp_ref[...] = (jnp.dot(h, wb_ref[...], preferred_element_type=jnp.float32)
                    + bb_ref[...])


_BN = 2000


def _dense_tc(partials, wg_t, bg, wb_t, bb):
    grid = N // _BN
    return pl.pallas_call(
        _dense_body,
        grid=(grid,),
        in_specs=[
            pl.BlockSpec((2, _BN, D), lambda i: (0, i, 0)),
            pl.BlockSpec((D, D), lambda i: (0, 0)),
            pl.BlockSpec((1, D), lambda i: (0, 0)),
            pl.BlockSpec((D, D), lambda i: (0, 0)),
            pl.BlockSpec((1, D), lambda i: (0, 0)),
        ],
        out_specs=[
            pl.BlockSpec((_BN, D), lambda i: (i, 0)),
            pl.BlockSpec((_BN, D), lambda i: (i, 0)),
        ],
        out_shape=[
            jax.ShapeDtypeStruct((N, D), jnp.float32),
            jax.ShapeDtypeStruct((N, D), jnp.float32),
        ],
    )(partials, wg_t, bg, wb_t, bb)


def kernel(edge_index, edge_values, emb_user, emb_fakers, emb_item,
           gc_w0, gc_b0, gc_w1, gc_b1, bi_w0, bi_b0, bi_w1, bi_b1):
    emb0 = jnp.concatenate([emb_user, emb_fakers, emb_item], axis=0)
    pad = EPAD - E
    packed = edge_index[1] + edge_index[0] * PACK
    packed = jnp.concatenate([packed, jnp.zeros((pad,), jnp.int32)])
    vals = jnp.concatenate([edge_values, jnp.zeros((pad,), jnp.float32)])

    def layer(emb, wg, bg, wb, bb):
        partials = _spmm_sc(emb, packed, vals).reshape(NC, N, D)
        return _dense_tc(partials, wg.T, bg.reshape(1, D),
                         wb.T, bb.reshape(1, D))

    emb1, mlp1 = layer(emb0, gc_w0, gc_b0, bi_w0, bi_b0)
    emb2, mlp2 = layer(emb1, gc_w1, gc_b1, bi_w1, bi_b1)

    all_emb = jnp.concatenate([emb0, mlp1, mlp2], axis=1)
    n_users_total = 5200
    return (all_emb[:n_users_total], all_emb[n_users_total:])


# speedup vs baseline: 2.1147x; 2.1147x over previous
import jax
import jax.numpy as jnp
from jax import lax
from jax.experimental import pallas as pl
from jax.experimental.pallas import tpu as pltpu
from jax.experimental.pallas import tpu_sc as plsc

N = 10000
D = 128
E = 320000
NC = 2
NS = 16
NW = NC * NS
CH = 80
NCHUNK = 126
EPW = CH * NCHUNK
EPAD = NW * EPW
NBUF = 2
NZB = N // CH
HF = CH // 2


def _spmm_body(emb_hbm, src_hbm, dst_hbm, vals_hbm, out_hbm,
               src_v, dst_v, vals_v, rows_v, dstc_v, acc_sh,
               g0, g1, h0, h1, s0, s1):
    gsem = (g0, g1)
    hsem = (h0, h1)
    ssem = (s0, s1)
    cid = lax.axis_index("c")
    sid = lax.axis_index("s")
    wid = cid * NS + sid
    ebase = wid * EPW

    zeros16 = jnp.zeros((16,), jnp.float32)
    zbuf = rows_v.at[0]

    @pl.loop(0, CH)
    def _zero_rows(i):
        for j in range(D // 16):
            zbuf[i, pl.ds(j * 16, 16)] = zeros16

    @pl.loop(sid, NZB, step=NS)
    def _zero_acc(cz):
        pltpu.sync_copy(zbuf, acc_sh.at[pl.ds(cz * CH, CH)])

    plsc.subcore_barrier()

    pltpu.sync_copy(src_hbm.at[pl.ds(ebase, EPW)], src_v)
    pltpu.sync_copy(dst_hbm.at[pl.ds(ebase, EPW)], dst_v)
    pltpu.sync_copy(vals_hbm.at[pl.ds(ebase, EPW)], vals_v)

    def stage_dstc(ch, b):
        for m in range(CH // 16):
            dstc_v[b, pl.ds(m * 16, 16)] = dst_v[pl.ds(ch * CH + m * 16, 16)]

    def issue_gather(ch, b):
        pltpu.async_copy(emb_hbm.at[src_v.at[pl.ds(ch * CH, HF)]],
                         rows_v.at[b, pl.ds(0, HF)], gsem[b])
        pltpu.async_copy(emb_hbm.at[src_v.at[pl.ds(ch * CH + HF, HF)]],
                         rows_v.at[b, pl.ds(HF, HF)], hsem[b])

    def wait_gather(b):
        pltpu.make_async_copy(emb_hbm.at[pl.ds(0, HF)],
                              rows_v.at[b, pl.ds(0, HF)], gsem[b]).wait()
        pltpu.make_async_copy(emb_hbm.at[pl.ds(0, HF)],
                              rows_v.at[b, pl.ds(HF, HF)], hsem[b]).wait()

    def wait_scatter(b):
        pltpu.make_async_copy(rows_v.at[b], acc_sh.at[pl.ds(0, CH)],
                              ssem[b]).wait()

    stage_dstc(0, 0)
    issue_gather(0, 0)

    @pl.loop(0, NCHUNK, step=NBUF)
    def _chunks(g):
        for b in range(NBUF):
            ch = g + b
            b1 = (b + 1) % NBUF

            @pl.when(ch + 1 < NCHUNK)
            def _prefetch():
                @pl.when(ch >= 1)
                def _drain():
                    wait_scatter(b1)

                stage_dstc(ch + 1, b1)
                issue_gather(ch + 1, b1)

            wait_gather(b)
            rowsb = rows_v.at[b]

            @pl.loop(0, CH // 16)
            def _scale(grp):
                v16 = vals_v[pl.ds(ch * CH + grp * 16, 16)]
                for r in range(16):
                    v = v16[r]
                    row = grp * 16 + r
                    for j in range(D // 16):
                        rowsb[row, pl.ds(j * 16, 16)] = (
                            rowsb[row, pl.ds(j * 16, 16)] * v)

            pltpu.async_copy(rows_v.at[b], acc_sh.at[dstc_v.at[b]],
                             ssem[b], add=True)

    for k in range(NBUF):
        wait_scatter((NCHUNK - NBUF + k) % NBUF)

    plsc.subcore_barrier()

    @pl.loop(sid, NZB, step=NS)
    def _writeout(co):
        r0 = co * CH
        pltpu.sync_copy(acc_sh.at[pl.ds(r0, CH)], zbuf)
        pltpu.sync_copy(zbuf, out_hbm.at[pl.ds(cid * N + r0, CH)])


_spmm_sc = pl.kernel(
    _spmm_body,
    out_type=jax.ShapeDtypeStruct((NC * N, D), jnp.float32),
    mesh=plsc.VectorSubcoreMesh(core_axis_name="c", subcore_axis_name="s"),
    scratch_types=[
        pltpu.VMEM((EPW,), jnp.int32),
        pltpu.VMEM((EPW,), jnp.int32),
        pltpu.VMEM((EPW,), jnp.float32),
        pltpu.VMEM((NBUF, CH, D), jnp.float32),
        pltpu.VMEM((NBUF, CH), jnp.int32),
        pltpu.VMEM_SHARED((N, D), jnp.float32),
        pltpu.SemaphoreType.DMA, pltpu.SemaphoreType.DMA,
        pltpu.SemaphoreType.DMA, pltpu.SemaphoreType.DMA,
        pltpu.SemaphoreType.DMA, pltpu.SemaphoreType.DMA,
    ],
)


def _dense_body(p_ref, wg_ref, bg_ref, wb_ref, bb_ref, emb_ref, mlp_ref):
    s = p_ref[0] + p_ref[1]
    h = jnp.dot(s, wg_ref[...], preferred_element_type=jnp.float32)
    h = h + bg_ref[...]
    h = jnp.where(h >= 0, h, 0.01 * h)
    emb_ref[...] = h
    mlp_ref[...] = (jnp.dot(h, wb_ref[...], preferred_element_type=jnp.float32)
                    + bb_ref[...])


_BN = 2000


def _dense_tc(partials, wg_t, bg, wb_t, bb):
    grid = N // _BN
    return pl.pallas_call(
        _dense_body,
        grid=(grid,),
        in_specs=[
            pl.BlockSpec((2, _BN, D), lambda i: (0, i, 0)),
            pl.BlockSpec((D, D), lambda i: (0, 0)),
            pl.BlockSpec((1, D), lambda i: (0, 0)),
            pl.BlockSpec((D, D), lambda i: (0, 0)),
            pl.BlockSpec((1, D), lambda i: (0, 0)),
        ],
        out_specs=[
            pl.BlockSpec((_BN, D), lambda i: (i, 0)),
            pl.BlockSpec((_BN, D), lambda i: (i, 0)),
        ],
        out_shape=[
            jax.ShapeDtypeStruct((N, D), jnp.float32),
            jax.ShapeDtypeStruct((N, D), jnp.float32),
        ],
    )(partials, wg_t, bg, wb_t, bb)


def kernel(edge_index, edge_values, emb_user, emb_fakers, emb_item,
           gc_w0, gc_b0, gc_w1, gc_b1, bi_w0, bi_b0, bi_w1, bi_b1):
    emb0 = jnp.concatenate([emb_user, emb_fakers, emb_item], axis=0)
    pad = EPAD - E
    dst = jnp.concatenate([edge_index[0], jnp.zeros((pad,), jnp.int32)])
    src = jnp.concatenate([edge_index[1], jnp.zeros((pad,), jnp.int32)])
    vals = jnp.concatenate([edge_values, jnp.zeros((pad,), jnp.float32)])

    def layer(emb, wg, bg, wb, bb):
        partials = _spmm_sc(emb, src, dst, vals).reshape(NC, N, D)
        return _dense_tc(partials, wg.T, bg.reshape(1, D),
                         wb.T, bb.reshape(1, D))

    emb1, mlp1 = layer(emb0, gc_w0, gc_b0, bi_w0, bi_b0)
    emb2, mlp2 = layer(emb1, gc_w1, gc_b1, bi_w1, bi_b1)

    all_emb = jnp.concatenate([emb0, mlp1, mlp2], axis=1)
    n_users_total = 5200
    return (all_emb[:n_users_total], all_emb[n_users_total:])
